# Initial kernel scaffold; baseline (speedup 1.0000x reference)
import jax
import jax.numpy as jnp
from jax.experimental import pallas as pl


def kernel(x_plate, edge_index, params):
    raise NotImplementedError("write your pallas kernel here")


# speedup vs baseline: 1.2632x; 1.2632x over previous
import functools

import numpy as np
import jax
import jax.numpy as jnp
from jax.experimental import pallas as pl
from jax.experimental.pallas import tpu as pltpu

_N = 4096
_E = 131072
_EMB = 128
_H = 4
_DH = _EMB // _H
_NUM_LAYERS = 2

_BLK = 256
_NBLK = _N // _BLK


def _attn_body(q_ref, k_ref, v_ref, wmean_ref, ao_ref, wbar_ref):
    i = pl.program_id(0)

    @pl.when(i == 0)
    def _():
        wbar_ref[...] = jnp.zeros_like(wbar_ref)

    q = q_ref[...]
    k = k_ref[...]
    scale = 1.0 / np.sqrt(_DH)
    wsum = jnp.zeros((_BLK, _N), jnp.float32)
    for h in range(_H):
        qh = q[:, h * _DH:(h + 1) * _DH] * scale
        kh = k[:, h * _DH:(h + 1) * _DH]
        s = jax.lax.dot_general(qh, kh, (((1,), (1,)), ((), ())),
                                preferred_element_type=jnp.float32)
        m = jnp.max(s, axis=-1, keepdims=True)
        e = jnp.exp(s - m)
        den = jnp.sum(e, axis=-1, keepdims=True)
        w = e / den
        wsum = wsum + w
        wbar_ref[h:h + 1, :] += jnp.sum(w, axis=0, keepdims=True)
    wmean_ref[...] = wsum * (1.0 / _H)

    @pl.when(i == _NBLK - 1)
    def _():
        v = v_ref[...]
        wb = wbar_ref[...] * (1.0 / _N)
        aos = []
        for h in range(_H):
            vh = v[:, h * _DH:(h + 1) * _DH]
            aos.append(jax.lax.dot_general(wb[h:h + 1, :], vh,
                                           (((1,), (0,)), ((), ())),
                                           preferred_element_type=jnp.float32))
        ao_ref[...] = jnp.concatenate(aos, axis=-1)


def _fused_attention(q, k, v):
    return pl.pallas_call(
        _attn_body,
        grid=(_NBLK,),
        in_specs=[
            pl.BlockSpec((_BLK, _EMB), lambda i: (i, 0)),
            pl.BlockSpec((_N, _EMB), lambda i: (0, 0)),
            pl.BlockSpec((_N, _EMB), lambda i: (0, 0)),
        ],
        out_specs=[
            pl.BlockSpec((_BLK, _N), lambda i: (i, 0)),
            pl.BlockSpec((1, _EMB), lambda i: (0, 0)),
        ],
        out_shape=[
            jax.ShapeDtypeStruct((_N, _N), jnp.float32),
            jax.ShapeDtypeStruct((1, _EMB), jnp.float32),
        ],
        scratch_shapes=[pltpu.VMEM((_H, _N), jnp.float32)],
    )(q, k, v)


def kernel(x_plate, edge_index, params):
    src = edge_index[0]
    dst = edge_index[1]
    h = x_plate
    for i in range(_NUM_LAYERS):
        wk = params[f'Wk{i}'].reshape(-1, _H, _DH)
        wq = params[f'Wq{i}'].reshape(-1, _H, _DH)
        wv = params[f'Wv{i}'].reshape(-1, _H, _DH)
        bk = params[f'bk{i}'].reshape(_H, _DH)
        bq = params[f'bq{i}'].reshape(_H, _DH)
        bv = params[f'bv{i}'].reshape(_H, _DH)
        prel = (params[f'p_rel{i}'] / np.sqrt(_DH))[:, None]
        wk_eff = jnp.einsum('ihd,hde->ihe', wk, params[f'a_rel{i}']).reshape(-1, _EMB)
        bk_eff = jnp.einsum('hd,hde->he', bk, params[f'a_rel{i}']).reshape(_EMB)
        wv_eff = jnp.einsum('ihd,hde->ihe', wv, params[f'm_rel{i}']).reshape(-1, _EMB)
        bv_eff = jnp.einsum('hd,hde->he', bv, params[f'm_rel{i}']).reshape(_EMB)
        wq_eff = (wq * prel).reshape(-1, _EMB)
        bq_eff = (bq * prel).reshape(_EMB)

        k2 = h @ wk_eff + bk_eff
        q2 = h @ wq_eff + bq_eff
        v2 = h @ wv_eff + bv_eff

        alpha = (q2[dst] * k2[src]).reshape(_E, _H, _DH).sum(-1)
        ex = jnp.exp(alpha)
        den = jax.ops.segment_sum(ex, dst, num_segments=_N)
        numer = jax.ops.segment_sum(
            v2[src].reshape(_E, _H, _DH) * ex[..., None], dst, num_segments=_N)
        agg = (numer / (den[..., None] + 1e-16)).reshape(-1, _EMB)

        out = jax.nn.gelu(agg)
        out = out @ params[f'Wo{i}'] + params[f'bo{i}']
        beta = jax.nn.sigmoid(params[f'skip{i}'])
        h = beta * out + (1.0 - beta) * h
        h = jax.nn.elu(h)

    h_global = h.mean(axis=0)
    gctx = h_global @ params['Wg'] + params['bg']
    qkv = h @ params['Win_attn'].T + params['bin_attn']
    q_, k_, v_ = jnp.split(qkv, 3, axis=-1)

    wmean, ao_mean = _fused_attention(q_, k_, v_)

    attn_out_mean = ao_mean[0] @ params['Wout_attn'].T + params['bout_attn']
    attn_weights = wmean[None]
    combined = gctx + attn_out_mean
    source_policy = jax.nn.softmax(combined @ params['Wsrc'] + params['bsrc'], axis=-1)
    dest_policy = jax.nn.softmax(combined @ params['Wdst'] + params['bdst'], axis=-1)
    value = combined @ params['Wcrit'] + params['bcrit']
    return (source_policy, dest_policy, value, attn_weights)


# speedup vs baseline: 28.7597x; 22.7669x over previous
import dataclasses
import functools

import numpy as np
import jax
import jax.numpy as jnp
from jax import lax
from jax.experimental import pallas as pl
from jax.experimental.pallas import tpu as pltpu
from jax.experimental.pallas import tpu_sc as plsc

_N = 4096
_E = 131072
_EMB = 128
_H = 4
_DH = _EMB // _H
_NUM_LAYERS = 2

_NP8 = 512
_NC = 2
_NS = 16
_NW = _NC * _NS
_EPW = _E // _NW
_CH = 128
_NCHUNK = _EPW // _CH

_BLK = 256
_NBLK = _N // _BLK
_PBLK = 512
_NPBLK = _N // _PBLK


def _proj_body(h_ref, w_ref, b_ref, q_ref, kv_ref, hmean_ref, hsum_ref):
    i = pl.program_id(0)

    @pl.when(i == 0)
    def _():
        hsum_ref[...] = jnp.zeros_like(hsum_ref)

    h = h_ref[...]
    r = jnp.dot(h, w_ref[...], preferred_element_type=jnp.float32) + b_ref[...]
    q_ref[...] = r[:, :_EMB]
    kv_ref[...] = r[:, _EMB:]
    hsum_ref[...] += jnp.sum(h, axis=0, keepdims=True)

    @pl.when(i == _NPBLK - 1)
    def _():
        hmean_ref[...] = hsum_ref[...] * (1.0 / _N)


def _proj(h, w, b):
    return pl.pallas_call(
        _proj_body,
        grid=(_NPBLK,),
        in_specs=[
            pl.BlockSpec((_PBLK, _EMB), lambda i: (i, 0)),
            pl.BlockSpec((_EMB, 3 * _EMB), lambda i: (0, 0)),
            pl.BlockSpec((1, 3 * _EMB), lambda i: (0, 0)),
        ],
        out_specs=[
            pl.BlockSpec((_PBLK, _EMB), lambda i: (i, 0)),
            pl.BlockSpec((_PBLK, 2 * _EMB), lambda i: (i, 0)),
            pl.BlockSpec((1, _EMB), lambda i: (0, 0)),
        ],
        out_shape=[
            jax.ShapeDtypeStruct((_N, _EMB), jnp.float32),
            jax.ShapeDtypeStruct((_N, 2 * _EMB), jnp.float32),
            jax.ShapeDtypeStruct((1, _EMB), jnp.float32),
        ],
        scratch_shapes=[pltpu.VMEM((1, _EMB), jnp.float32)],
    )(h, w, b)


def _edge_body(q_hbm, kv_hbm, src_hbm, dst_hbm, z_hbm, outv_hbm, oute_hbm,
               sh_v, sh_e, sidx, didx, idx2, qr, kvr, ov, ov2, sem):
    cid = lax.axis_index("c")
    sid = lax.axis_index("s")
    wid = cid * _NS + sid
    stripe = _N // _NS
    estripe = _NP8 // _NS

    pltpu.sync_copy(z_hbm, sh_v.at[pl.ds(sid * stripe, stripe)])
    pltpu.sync_copy(z_hbm.at[pl.ds(0, estripe)],
                    sh_e.at[pl.ds(sid * estripe, estripe)])
    plsc.subcore_barrier()

    lane = lax.broadcasted_iota(jnp.int32, (16,), 0)
    zero16 = jnp.zeros((16,), jnp.float32)

    @pl.loop(0, _NCHUNK)
    def _chunk(c):
        base = wid * _EPW + c * _CH
        pltpu.sync_copy(src_hbm.at[pl.ds(base, _CH)], sidx)
        pltpu.sync_copy(dst_hbm.at[pl.ds(base, _CH)], didx)
        pltpu.async_copy(q_hbm.at[didx], qr, sem).wait()
        pltpu.async_copy(kv_hbm.at[sidx], kvr, sem).wait()

        for g in range(_CH // 16):
            idx2[pl.ds(g * 16, 16)] = lax.shift_right_arithmetic(
                didx[pl.ds(g * 16, 16)], 3)

        @pl.loop(0, _CH // 16)
        def _grp(g):
            dvec = didx[pl.ds(g * 16, 16)]
            for j in range(16):
                e = g * 16 + j
                exl = zero16
                for h in range(_H):
                    c0 = h * _DH
                    q0 = qr[e, pl.ds(c0, 16)]
                    q1 = qr[e, pl.ds(c0 + 16, 16)]
                    k0 = kvr[e, pl.ds(c0, 16)]
                    k1 = kvr[e, pl.ds(c0 + 16, 16)]
                    a = jnp.sum(q0 * k0 + q1 * k1)
                    ex = jnp.exp(jnp.full((16,), a, jnp.float32))
                    v0 = kvr[e, pl.ds(_EMB + c0, 16)]
                    v1 = kvr[e, pl.ds(_EMB + c0 + 16, 16)]
                    ov[e, pl.ds(c0, 16)] = v0 * ex
                    ov[e, pl.ds(c0 + 16, 16)] = v1 * ex
                    exl = exl + jnp.where(lane == h, ex, 0.0)
                off = (dvec[j] & 7) * 16
                for jj in range(8):
                    ov2[e, pl.ds(jj * 16, 16)] = zero16
                ov2[e, pl.ds(off, 16)] = exl

        pltpu.sync_copy(ov, sh_v.at[didx], add=True)
        pltpu.sync_copy(ov2, sh_e.at[idx2], add=True)

    plsc.subcore_barrier()
    pltpu.sync_copy(sh_v.at[pl.ds(sid * stripe, stripe)],
                    outv_hbm.at[cid, pl.ds(sid * stripe, stripe)])
    pltpu.sync_copy(sh_e.at[pl.ds(sid * estripe, estripe)],
                    oute_hbm.at[cid, pl.ds(sid * estripe, estripe)])


def _edge_stage(q2, kv2, src, dst, zrows):
    mesh = plsc.VectorSubcoreMesh(core_axis_name="c", subcore_axis_name="s",
                                  num_cores=_NC, num_subcores=_NS)
    cp = pltpu.CompilerParams()
    if "needs_layout_passes" in pltpu.CompilerParams.__dataclass_fields__:
        cp = dataclasses.replace(cp, needs_layout_passes=False)
    run = pl.kernel(
        _edge_body,
        mesh=mesh,
        compiler_params=cp,
        out_type=[
            jax.ShapeDtypeStruct((_NC, _N, _EMB), jnp.float32),
            jax.ShapeDtypeStruct((_NC, _NP8, _EMB), jnp.float32),
        ],
        scratch_types=[
            pltpu.VMEM_SHARED((_N, _EMB), jnp.float32),
            pltpu.VMEM_SHARED((_NP8, _EMB), jnp.float32),
            pltpu.VMEM((_CH,), jnp.int32),
            pltpu.VMEM((_CH,), jnp.int32),
            pltpu.VMEM((_CH,), jnp.int32),
            pltpu.VMEM((_CH, _EMB), jnp.float32),
            pltpu.VMEM((_CH, 2 * _EMB), jnp.float32),
            pltpu.VMEM((_CH, _EMB), jnp.float32),
            pltpu.VMEM((_CH, _EMB), jnp.float32),
            pltpu.SemaphoreType.DMA,
        ],
    )
    return run(q2, kv2, src, dst, zrows)


def _combine_body(p_ref, den4_ref, h_ref, wo_ref, bo_ref, beta_ref, o_ref):
    p = p_ref[0] + p_ref[1]
    row = lax.broadcasted_iota(jnp.int32, (_H, _EMB), 0)
    col = lax.broadcasted_iota(jnp.int32, (_H, _EMB), 1)
    expand = (row == col // _DH).astype(jnp.float32)
    den = jnp.dot(den4_ref[...], expand, preferred_element_type=jnp.float32)
    agg = p / (den + 1e-16)
    out = jax.nn.gelu(agg)
    out = jnp.dot(out, wo_ref[...], preferred_element_type=jnp.float32) + bo_ref[...]
    beta = beta_ref[0, 0]
    x = beta * out + (1.0 - beta) * h_ref[...]
    o_ref[...] = jnp.where(x > 0, x, jnp.exp(jnp.minimum(x, 0.0)) - 1.0)


def _combine(parts, den4, h, wo, bo, beta):
    return pl.pallas_call(
        _combine_body,
        grid=(_NPBLK,),
        in_specs=[
            pl.BlockSpec((_NC, _PBLK, _EMB), lambda i: (0, i, 0)),
            pl.BlockSpec((_PBLK, _H), lambda i: (i, 0)),
            pl.BlockSpec((_PBLK, _EMB), lambda i: (i, 0)),
            pl.BlockSpec((_EMB, _EMB), lambda i: (0, 0)),
            pl.BlockSpec((1, _EMB), lambda i: (0, 0)),
            pl.BlockSpec((1, 1), lambda i: (0, 0)),
        ],
        out_specs=pl.BlockSpec((_PBLK, _EMB), lambda i: (i, 0)),
        out_shape=jax.ShapeDtypeStruct((_N, _EMB), jnp.float32),
    )(parts, den4, h, wo, bo, beta)


def _attn_body(q_ref, kv_ref, wmean_ref, ao_ref, wbar_ref):
    i = pl.program_id(0)

    @pl.when(i == 0)
    def _():
        wbar_ref[...] = jnp.zeros_like(wbar_ref)

    q = q_ref[...]
    k = kv_ref[:, :_EMB]
    scale = 1.0 / np.sqrt(_DH)
    wsum = jnp.zeros((_BLK, _N), jnp.float32)
    for h in range(_H):
        qh = q[:, h * _DH:(h + 1) * _DH] * scale
        kh = k[:, h * _DH:(h + 1) * _DH]
        s = lax.dot_general(qh, kh, (((1,), (1,)), ((), ())),
                            preferred_element_type=jnp.float32)
        m = jnp.max(s, axis=-1, keepdims=True)
        e = jnp.exp(s - m)
        den = jnp.sum(e, axis=-1, keepdims=True)
        w = e / den
        wsum = wsum + w
        wbar_ref[h:h + 1, :] += jnp.sum(w, axis=0, keepdims=True)
    wmean_ref[...] = wsum * (1.0 / _H)

    @pl.when(i == _NBLK - 1)
    def _():
        v = kv_ref[:, _EMB:]
        wb = wbar_ref[...] * (1.0 / _N)
        aos = []
        for h in range(_H):
            vh = v[:, h * _DH:(h + 1) * _DH]
            aos.append(lax.dot_general(wb[h:h + 1, :], vh,
                                       (((1,), (0,)), ((), ())),
                                       preferred_element_type=jnp.float32))
        ao_ref[...] = jnp.concatenate(aos, axis=-1)


def _fused_attention(q, kv):
    return pl.pallas_call(
        _attn_body,
        grid=(_NBLK,),
        in_specs=[
            pl.BlockSpec((_BLK, _EMB), lambda i: (i, 0)),
            pl.BlockSpec((_N, 2 * _EMB), lambda i: (0, 0)),
        ],
        out_specs=[
            pl.BlockSpec((_BLK, _N), lambda i: (i, 0)),
            pl.BlockSpec((1, _EMB), lambda i: (0, 0)),
        ],
        out_shape=[
            jax.ShapeDtypeStruct((_N, _N), jnp.float32),
            jax.ShapeDtypeStruct((1, _EMB), jnp.float32),
        ],
        scratch_shapes=[pltpu.VMEM((_H, _N), jnp.float32)],
    )(q, kv)


def _head_body(hmean_ref, ao_ref, wg_ref, bg_ref, woutT_ref, bout_ref,
               wsrc_ref, bsrc_ref, wdst_ref, bdst_ref, wcrit_ref, bcrit_ref,
               sp_ref, dp_ref, val_ref):
    gctx = jnp.dot(hmean_ref[...], wg_ref[...],
                   preferred_element_type=jnp.float32) + bg_ref[...]
    aom = jnp.dot(ao_ref[...], woutT_ref[...],
                  preferred_element_type=jnp.float32) + bout_ref[...]
    combined = gctx + aom

    def _softmax(x):
        m = jnp.max(x, axis=-1, keepdims=True)
        e = jnp.exp(x - m)
        return e / jnp.sum(e, axis=-1, keepdims=True)

    sp_ref[...] = _softmax(jnp.dot(combined, wsrc_ref[...],
                                   preferred_element_type=jnp.float32) + bsrc_ref[...])
    dp_ref[...] = _softmax(jnp.dot(combined, wdst_ref[...],
                                   preferred_element_type=jnp.float32) + bdst_ref[...])
    val_ref[...] = jnp.dot(combined, wcrit_ref[...],
                           preferred_element_type=jnp.float32) + bcrit_ref[...]


def _head(hmean, ao, wg, bg, woutT, bout, wsrc, bsrc, wdst, bdst, wcrit, bcrit):
    return pl.pallas_call(
        _head_body,
        out_shape=[
            jax.ShapeDtypeStruct((1, 4), jnp.float32),
            jax.ShapeDtypeStruct((1, 4), jnp.float32),
            jax.ShapeDtypeStruct((1, 1), jnp.float32),
        ],
    )(hmean, ao, wg, bg, woutT, bout, wsrc, bsrc, wdst, bdst, wcrit, bcrit)


def kernel(x_plate, edge_index, params):
    src = edge_index[0]
    dst = edge_index[1]
    zrows = jnp.zeros((_N // _NS, _EMB), jnp.float32)

    h = x_plate
    for i in range(_NUM_LAYERS):
        wk = params[f'Wk{i}'].reshape(-1, _H, _DH)
        wq = params[f'Wq{i}'].reshape(-1, _H, _DH)
        wv = params[f'Wv{i}'].reshape(-1, _H, _DH)
        bk = params[f'bk{i}'].reshape(_H, _DH)
        bq = params[f'bq{i}'].reshape(_H, _DH)
        bv = params[f'bv{i}'].reshape(_H, _DH)
        prel = (params[f'p_rel{i}'] / np.sqrt(_DH))[:, None]
        wk_eff = jnp.einsum('ihd,hde->ihe', wk, params[f'a_rel{i}']).reshape(-1, _EMB)
        bk_eff = jnp.einsum('hd,hde->he', bk, params[f'a_rel{i}']).reshape(_EMB)
        wv_eff = jnp.einsum('ihd,hde->ihe', wv, params[f'm_rel{i}']).reshape(-1, _EMB)
        bv_eff = jnp.einsum('hd,hde->he', bv, params[f'm_rel{i}']).reshape(_EMB)
        wq_eff = (wq * prel).reshape(-1, _EMB)
        bq_eff = (bq * prel).reshape(_EMB)

        w_all = jnp.concatenate([wq_eff, wk_eff, wv_eff], axis=1)
        b_all = jnp.concatenate([bq_eff, bk_eff, bv_eff])[None]

        q2, kv2, _ = _proj(h, w_all, b_all)
        parts_v, parts_e = _edge_stage(q2, kv2, src, dst, zrows)
        den4 = (parts_e.sum(axis=0)
                .reshape(_NP8, 8, 16)[:, :, :_H].reshape(_N, _H))
        beta = jax.nn.sigmoid(params[f'skip{i}']).reshape(1, 1)
        h = _combine(parts_v, den4, h, params[f'Wo{i}'], params[f'bo{i}'][None],
                     beta)

    w_in = params['Win_attn'].T
    b_in = params['bin_attn'][None]
    q_, kv_, hmean = _proj(h, w_in, b_in)

    wmean, ao_mean = _fused_attention(q_, kv_)

    sp, dp, val = _head(
        hmean, ao_mean,
        params['Wg'], params['bg'][None],
        params['Wout_attn'].T, params['bout_attn'][None],
        params['Wsrc'], params['bsrc'][None],
        params['Wdst'], params['bdst'][None],
        params['Wcrit'], params['bcrit'][None],
    )
    return (sp[0], dp[0], val[0], wmean[None])


# speedup vs baseline: 35.7836x; 1.2442x over previous
import dataclasses
import functools

import numpy as np
import jax
import jax.numpy as jnp
from jax import lax
from jax.experimental import pallas as pl
from jax.experimental.pallas import tpu as pltpu
from jax.experimental.pallas import tpu_sc as plsc

_N = 4096
_E = 131072
_EMB = 128
_H = 4
_DH = _EMB // _H
_NUM_LAYERS = 2

_NP8 = 512
_NC = 2
_NS = 16
_NW = _NC * _NS
_EPW = _E // _NW
_CH = 64
_NCHUNK = _EPW // _CH
_NPAIR = _NCHUNK // 2

_BLK = 256
_NBLK = _N // _BLK
_PBLK = 512
_NPBLK = _N // _PBLK


def _proj_body(h_ref, w_ref, b_ref, q_ref, kv_ref, hmean_ref, hsum_ref):
    i = pl.program_id(0)

    @pl.when(i == 0)
    def _():
        hsum_ref[...] = jnp.zeros_like(hsum_ref)

    h = h_ref[...]
    r = jnp.dot(h, w_ref[...], preferred_element_type=jnp.float32) + b_ref[...]
    q_ref[...] = r[:, :_EMB]
    kv_ref[...] = r[:, _EMB:]
    hsum_ref[...] += jnp.sum(h, axis=0, keepdims=True)

    @pl.when(i == _NPBLK - 1)
    def _():
        hmean_ref[...] = hsum_ref[...] * (1.0 / _N)


def _proj(h, w, b):
    return pl.pallas_call(
        _proj_body,
        grid=(_NPBLK,),
        in_specs=[
            pl.BlockSpec((_PBLK, _EMB), lambda i: (i, 0)),
            pl.BlockSpec((_EMB, 3 * _EMB), lambda i: (0, 0)),
            pl.BlockSpec((1, 3 * _EMB), lambda i: (0, 0)),
        ],
        out_specs=[
            pl.BlockSpec((_PBLK, _EMB), lambda i: (i, 0)),
            pl.BlockSpec((_PBLK, 2 * _EMB), lambda i: (i, 0)),
            pl.BlockSpec((1, _EMB), lambda i: (0, 0)),
        ],
        out_shape=[
            jax.ShapeDtypeStruct((_N, _EMB), jnp.float32),
            jax.ShapeDtypeStruct((_N, 2 * _EMB), jnp.float32),
            jax.ShapeDtypeStruct((1, _EMB), jnp.float32),
        ],
        scratch_shapes=[pltpu.VMEM((1, _EMB), jnp.float32)],
    )(h, w, b)


def _edge_body(q_hbm, kv_hbm, src_hbm, dst_hbm, z_hbm, outv_hbm, oute_hbm,
               sh_v, sh_e, sidx_all, didx_all,
               qr0, kvr0, ov0, ov20, didxc0, idx20,
               qr1, kvr1, ov1, ov21, didxc1, idx21,
               sg0, sg1, ss0, ss1):
    cid = lax.axis_index("c")
    sid = lax.axis_index("s")
    wid = cid * _NS + sid
    stripe = _N // _NS
    estripe = _NP8 // _NS

    pltpu.sync_copy(z_hbm, sh_v.at[pl.ds(sid * stripe, stripe)])
    pltpu.sync_copy(z_hbm.at[pl.ds(0, estripe)],
                    sh_e.at[pl.ds(sid * estripe, estripe)])
    plsc.subcore_barrier()

    lane = lax.broadcasted_iota(jnp.int32, (16,), 0)
    zero16 = jnp.zeros((16,), jnp.float32)
    zero16i = jnp.zeros((16,), jnp.int32)

    pltpu.sync_copy(src_hbm.at[pl.ds(wid * _EPW, _EPW)], sidx_all)
    pltpu.sync_copy(dst_hbm.at[pl.ds(wid * _EPW, _EPW)], didx_all)

    bufs = ((qr0, kvr0, ov0, ov20, didxc0, idx20, sg0, ss0),
            (qr1, kvr1, ov1, ov21, didxc1, idx21, sg1, ss1))

    def gathers(c, qr, kvr, sem):
        off = c * _CH
        return (pltpu.make_async_copy(q_hbm.at[didx_all.at[pl.ds(off, _CH)]],
                                      qr, sem),
                pltpu.make_async_copy(kv_hbm.at[sidx_all.at[pl.ds(off, _CH)]],
                                      kvr, sem))

    def scatters(ov, ov2, didxc, idx2, sem):
        return (pltpu.make_async_copy(ov, sh_v.at[didxc], sem),
                pltpu.make_async_copy(ov2, sh_e.at[idx2], sem))

    def issue(pair, add=False):
        for cp_ in pair:
            cp_.start(add=add)

    def wait(pair):
        for cp_ in pair:
            cp_.wait()

    for (qr, kvr, ov, ov2, didxc, idx2, sg, ss) in bufs:
        @pl.loop(0, _CH)
        def _z(e):
            for j in range(0, _EMB, 16):
                ov[e, pl.ds(j, 16)] = zero16
                ov2[e, pl.ds(j, 16)] = zero16
        for g in range(_CH // 16):
            didxc[pl.ds(g * 16, 16)] = zero16i
            idx2[pl.ds(g * 16, 16)] = zero16i
        issue(scatters(ov, ov2, didxc, idx2, ss), add=True)
    issue(gathers(0, qr0, kvr0, sg0))

    def compute(c, qr, kvr, ov, ov2, didxc, idx2):
        off = c * _CH
        for gi in range(_CH // 16):
            dv = didx_all[pl.ds(off + gi * 16, 16)]
            didxc[pl.ds(gi * 16, 16)] = dv
            idx2[pl.ds(gi * 16, 16)] = lax.shift_right_arithmetic(dv, 3)
            for j in range(16):
                e = gi * 16 + j
                exl = zero16
                for h in range(_H):
                    c0 = h * _DH
                    q0 = qr[e, pl.ds(c0, 16)]
                    q1 = qr[e, pl.ds(c0 + 16, 16)]
                    k0 = kvr[e, pl.ds(c0, 16)]
                    k1 = kvr[e, pl.ds(c0 + 16, 16)]
                    a = jnp.sum(q0 * k0 + q1 * k1)
                    ex = jnp.exp(jnp.full((16,), a, jnp.float32))
                    v0 = kvr[e, pl.ds(_EMB + c0, 16)]
                    v1 = kvr[e, pl.ds(_EMB + c0 + 16, 16)]
                    ov[e, pl.ds(c0, 16)] = v0 * ex
                    ov[e, pl.ds(c0 + 16, 16)] = v1 * ex
                    exl = exl + jnp.where(lane == h, ex, 0.0)
                o2 = (dv[j] & 7) * 16
                for jj in range(8):
                    ov2[e, pl.ds(jj * 16, 16)] = zero16
                ov2[e, pl.ds(o2, 16)] = exl

    @pl.loop(0, _NPAIR)
    def _pair(g):
        c0 = 2 * g
        c1 = c0 + 1
        issue(gathers(c1, qr1, kvr1, sg1))
        wait(gathers(c0, qr0, kvr0, sg0))
        wait(scatters(ov0, ov20, didxc0, idx20, ss0))
        compute(c0, qr0, kvr0, ov0, ov20, didxc0, idx20)
        issue(scatters(ov0, ov20, didxc0, idx20, ss0), add=True)
        cn = jnp.minimum(c0 + 2, _NCHUNK - 1)
        issue(gathers(cn, qr0, kvr0, sg0))
        wait(gathers(c1, qr1, kvr1, sg1))
        wait(scatters(ov1, ov21, didxc1, idx21, ss1))
        compute(c1, qr1, kvr1, ov1, ov21, didxc1, idx21)
        issue(scatters(ov1, ov21, didxc1, idx21, ss1), add=True)

    wait(gathers(_NCHUNK - 1, qr0, kvr0, sg0))
    wait(scatters(ov0, ov20, didxc0, idx20, ss0))
    wait(scatters(ov1, ov21, didxc1, idx21, ss1))

    plsc.subcore_barrier()
    pltpu.sync_copy(sh_v.at[pl.ds(sid * stripe, stripe)],
                    outv_hbm.at[cid, pl.ds(sid * stripe, stripe)])
    pltpu.sync_copy(sh_e.at[pl.ds(sid * estripe, estripe)],
                    oute_hbm.at[cid, pl.ds(sid * estripe, estripe)])


def _edge_stage(q2, kv2, src, dst, zrows):
    mesh = plsc.VectorSubcoreMesh(core_axis_name="c", subcore_axis_name="s",
                                  num_cores=_NC, num_subcores=_NS)
    cp = pltpu.CompilerParams()
    if "needs_layout_passes" in pltpu.CompilerParams.__dataclass_fields__:
        cp = dataclasses.replace(cp, needs_layout_passes=False)
    run = pl.kernel(
        _edge_body,
        mesh=mesh,
        compiler_params=cp,
        out_type=[
            jax.ShapeDtypeStruct((_NC, _N, _EMB), jnp.float32),
            jax.ShapeDtypeStruct((_NC, _NP8, _EMB), jnp.float32),
        ],
        scratch_types=(
            [
                pltpu.VMEM_SHARED((_N, _EMB), jnp.float32),
                pltpu.VMEM_SHARED((_NP8, _EMB), jnp.float32),
                pltpu.VMEM((_EPW,), jnp.int32),
                pltpu.VMEM((_EPW,), jnp.int32),
            ]
            + 2 * [
                pltpu.VMEM((_CH, _EMB), jnp.float32),
                pltpu.VMEM((_CH, 2 * _EMB), jnp.float32),
                pltpu.VMEM((_CH, _EMB), jnp.float32),
                pltpu.VMEM((_CH, _EMB), jnp.float32),
                pltpu.VMEM((_CH,), jnp.int32),
                pltpu.VMEM((_CH,), jnp.int32),
            ]
            + 4 * [pltpu.SemaphoreType.DMA]
        ),
    )
    return run(q2, kv2, src, dst, zrows)


def _combine_body(p_ref, den4_ref, h_ref, wo_ref, bo_ref, beta_ref, o_ref):
    p = p_ref[0] + p_ref[1]
    row = lax.broadcasted_iota(jnp.int32, (_H, _EMB), 0)
    col = lax.broadcasted_iota(jnp.int32, (_H, _EMB), 1)
    expand = (row == col // _DH).astype(jnp.float32)
    den = jnp.dot(den4_ref[...], expand, preferred_element_type=jnp.float32)
    agg = p / (den + 1e-16)
    out = jax.nn.gelu(agg)
    out = jnp.dot(out, wo_ref[...], preferred_element_type=jnp.float32) + bo_ref[...]
    beta = beta_ref[0, 0]
    x = beta * out + (1.0 - beta) * h_ref[...]
    o_ref[...] = jnp.where(x > 0, x, jnp.exp(jnp.minimum(x, 0.0)) - 1.0)


def _combine(parts, den4, h, wo, bo, beta):
    return pl.pallas_call(
        _combine_body,
        grid=(_NPBLK,),
        in_specs=[
            pl.BlockSpec((_NC, _PBLK, _EMB), lambda i: (0, i, 0)),
            pl.BlockSpec((_PBLK, _H), lambda i: (i, 0)),
            pl.BlockSpec((_PBLK, _EMB), lambda i: (i, 0)),
            pl.BlockSpec((_EMB, _EMB), lambda i: (0, 0)),
            pl.BlockSpec((1, _EMB), lambda i: (0, 0)),
            pl.BlockSpec((1, 1), lambda i: (0, 0)),
        ],
        out_specs=pl.BlockSpec((_PBLK, _EMB), lambda i: (i, 0)),
        out_shape=jax.ShapeDtypeStruct((_N, _EMB), jnp.float32),
    )(parts, den4, h, wo, bo, beta)


def _attn_body(q_ref, kv_ref, wmean_ref, ao_ref, wbar_ref):
    i = pl.program_id(0)

    @pl.when(i == 0)
    def _():
        wbar_ref[...] = jnp.zeros_like(wbar_ref)

    q = q_ref[...]
    k = kv_ref[:, :_EMB]
    scale = 1.0 / np.sqrt(_DH)
    wsum = jnp.zeros((_BLK, _N), jnp.float32)
    for h in range(_H):
        qh = q[:, h * _DH:(h + 1) * _DH] * scale
        kh = k[:, h * _DH:(h + 1) * _DH]
        s = lax.dot_general(qh, kh, (((1,), (1,)), ((), ())),
                            preferred_element_type=jnp.float32)
        m = jnp.max(s, axis=-1, keepdims=True)
        e = jnp.exp(s - m)
        den = jnp.sum(e, axis=-1, keepdims=True)
        w = e / den
        wsum = wsum + w
        wbar_ref[h:h + 1, :] += jnp.sum(w, axis=0, keepdims=True)
    wmean_ref[...] = wsum * (1.0 / _H)

    @pl.when(i == _NBLK - 1)
    def _():
        v = kv_ref[:, _EMB:]
        wb = wbar_ref[...] * (1.0 / _N)
        aos = []
        for h in range(_H):
            vh = v[:, h * _DH:(h + 1) * _DH]
            aos.append(lax.dot_general(wb[h:h + 1, :], vh,
                                       (((1,), (0,)), ((), ())),
                                       preferred_element_type=jnp.float32))
        ao_ref[...] = jnp.concatenate(aos, axis=-1)


def _fused_attention(q, kv):
    return pl.pallas_call(
        _attn_body,
        grid=(_NBLK,),
        in_specs=[
            pl.BlockSpec((_BLK, _EMB), lambda i: (i, 0)),
            pl.BlockSpec((_N, 2 * _EMB), lambda i: (0, 0)),
        ],
        out_specs=[
            pl.BlockSpec((_BLK, _N), lambda i: (i, 0)),
            pl.BlockSpec((1, _EMB), lambda i: (0, 0)),
        ],
        out_shape=[
            jax.ShapeDtypeStruct((_N, _N), jnp.float32),
            jax.ShapeDtypeStruct((1, _EMB), jnp.float32),
        ],
        scratch_shapes=[pltpu.VMEM((_H, _N), jnp.float32)],
    )(q, kv)


def _head_body(hmean_ref, ao_ref, wg_ref, bg_ref, woutT_ref, bout_ref,
               wsrc_ref, bsrc_ref, wdst_ref, bdst_ref, wcrit_ref, bcrit_ref,
               sp_ref, dp_ref, val_ref):
    gctx = jnp.dot(hmean_ref[...], wg_ref[...],
                   preferred_element_type=jnp.float32) + bg_ref[...]
    aom = jnp.dot(ao_ref[...], woutT_ref[...],
                  preferred_element_type=jnp.float32) + bout_ref[...]
    combined = gctx + aom

    def _softmax(x):
        m = jnp.max(x, axis=-1, keepdims=True)
        e = jnp.exp(x - m)
        return e / jnp.sum(e, axis=-1, keepdims=True)

    sp_ref[...] = _softmax(jnp.dot(combined, wsrc_ref[...],
                                   preferred_element_type=jnp.float32) + bsrc_ref[...])
    dp_ref[...] = _softmax(jnp.dot(combined, wdst_ref[...],
                                   preferred_element_type=jnp.float32) + bdst_ref[...])
    val_ref[...] = jnp.dot(combined, wcrit_ref[...],
                           preferred_element_type=jnp.float32) + bcrit_ref[...]


def _head(hmean, ao, wg, bg, woutT, bout, wsrc, bsrc, wdst, bdst, wcrit, bcrit):
    return pl.pallas_call(
        _head_body,
        out_shape=[
            jax.ShapeDtypeStruct((1, 4), jnp.float32),
            jax.ShapeDtypeStruct((1, 4), jnp.float32),
            jax.ShapeDtypeStruct((1, 1), jnp.float32),
        ],
    )(hmean, ao, wg, bg, woutT, bout, wsrc, bsrc, wdst, bdst, wcrit, bcrit)


def kernel(x_plate, edge_index, params):
    src = edge_index[0]
    dst = edge_index[1]
    zrows = jnp.zeros((_N // _NS, _EMB), jnp.float32)

    h = x_plate
    for i in range(_NUM_LAYERS):
        wk = params[f'Wk{i}'].reshape(-1, _H, _DH)
        wq = params[f'Wq{i}'].reshape(-1, _H, _DH)
        wv = params[f'Wv{i}'].reshape(-1, _H, _DH)
        bk = params[f'bk{i}'].reshape(_H, _DH)
        bq = params[f'bq{i}'].reshape(_H, _DH)
        bv = params[f'bv{i}'].reshape(_H, _DH)
        prel = (params[f'p_rel{i}'] / np.sqrt(_DH))[:, None]
        wk_eff = jnp.einsum('ihd,hde->ihe', wk, params[f'a_rel{i}']).reshape(-1, _EMB)
        bk_eff = jnp.einsum('hd,hde->he', bk, params[f'a_rel{i}']).reshape(_EMB)
        wv_eff = jnp.einsum('ihd,hde->ihe', wv, params[f'm_rel{i}']).reshape(-1, _EMB)
        bv_eff = jnp.einsum('hd,hde->he', bv, params[f'm_rel{i}']).reshape(_EMB)
        wq_eff = (wq * prel).reshape(-1, _EMB)
        bq_eff = (bq * prel).reshape(_EMB)

        w_all = jnp.concatenate([wq_eff, wk_eff, wv_eff], axis=1)
        b_all = jnp.concatenate([bq_eff, bk_eff, bv_eff])[None]

        q2, kv2, _ = _proj(h, w_all, b_all)
        parts_v, parts_e = _edge_stage(q2, kv2, src, dst, zrows)
        den4 = (parts_e.sum(axis=0)
                .reshape(_NP8, 8, 16)[:, :, :_H].reshape(_N, _H))
        beta = jax.nn.sigmoid(params[f'skip{i}']).reshape(1, 1)
        h = _combine(parts_v, den4, h, params[f'Wo{i}'], params[f'bo{i}'][None],
                     beta)

    w_in = params['Win_attn'].T
    b_in = params['bin_attn'][None]
    q_, kv_, hmean = _proj(h, w_in, b_in)

    wmean, ao_mean = _fused_attention(q_, kv_)

    sp, dp, val = _head(
        hmean, ao_mean,
        params['Wg'], params['bg'][None],
        params['Wout_attn'].T, params['bout_attn'][None],
        params['Wsrc'], params['bsrc'][None],
        params['Wdst'], params['bdst'][None],
        params['Wcrit'], params['bcrit'][None],
    )
    return (sp[0], dp[0], val[0], wmean[None])
